# Initial kernel scaffold; baseline (speedup 1.0000x reference)
import jax
import jax.numpy as jnp
from jax.experimental import pallas as pl


def kernel(pred0, pred1, target):
    raise NotImplementedError("write your pallas kernel here")


# speedup vs baseline: 1.2105x; 1.2105x over previous
import functools

import jax
import jax.numpy as jnp
from jax.experimental import pallas as pl
from jax.experimental.pallas import tpu as pltpu

_IGNORE_INDEX = 255
_THRESH = 0.7
_MIN_KEPT = 100000
_HIST_BINS = 64
_KEEP_ALL_THR = 1.5
_INVALID_PROB = 2.0
_MM_DTYPE = jnp.bfloat16
_TILE_H = 256
_VMEM_LIMIT = (64 << 20) * 3 // 4


def _cdiv(a, b):
    return -(-a // b)


def _round_up(a, b):
    return _cdiv(a, b) * b


def _interp_matrix(out_size, in_size):
    if out_size == 1:
        src = jnp.zeros((1,), jnp.float32)
    else:
        src = jnp.arange(out_size, dtype=jnp.float32) * (in_size - 1) / (out_size - 1)
    i0 = jnp.clip(jnp.floor(src).astype(jnp.int32), 0, in_size - 1)
    i1 = jnp.clip(i0 + 1, 0, in_size - 1)
    w1 = src - i0.astype(jnp.float32)
    w0 = 1.0 - w1
    cols = jnp.arange(in_size, dtype=jnp.int32)[None, :]
    mat = (w0[:, None] * (cols == i0[:, None]).astype(jnp.float32)
           + w1[:, None] * (cols == i1[:, None]).astype(jnp.float32))
    return mat


def _edges_tuple(thresh, nbins):
    step = (1.0 - float(thresh)) / (nbins - 1)
    return tuple([float(thresh) + j * step for j in range(nbins - 1)] + [1.0 + 1e-3])


def _sum2d(x):
    return jnp.sum(jnp.sum(x, axis=1, keepdims=True), axis=0, keepdims=True)


def _main_kernel(p0_ref, p1_ref, wh_ref, wwt_ref, tgt_ref,
                 ce_ref, prob_ref, stats_ref,
                 *, num_classes, w_pad, ignore_index, edges):
    C = num_classes
    wh = wh_ref[...]
    wwt = wwt_ref[...]
    tgt = tgt_ref[0]
    p0 = p0_ref[0]
    p1 = p1_ref[0]

    mm = lambda a, b: jnp.dot(a, b, preferred_element_type=jnp.float32)

    m0c = p0[:, :w_pad]
    m1c = p1[:, :w_pad]
    for c in range(1, C):
        m0c = jnp.maximum(m0c, p0[:, c * w_pad:(c + 1) * w_pad])
        m1c = jnp.maximum(m1c, p1[:, c * w_pad:(c + 1) * w_pad])

    t0 = mm(wh, jnp.concatenate([p0, m0c], axis=1)).astype(_MM_DTYPE)
    t1 = mm(wh, jnp.concatenate([p1, m1c], axis=1)).astype(_MM_DTYPE)

    cu = lambda t, c: mm(t[:, c * w_pad:(c + 1) * w_pad], wwt)

    m0 = cu(t0, C)
    m1 = cu(t1, C)

    se0 = jnp.zeros_like(m0)
    se1 = jnp.zeros_like(m1)
    gt0 = jnp.zeros_like(m0)
    gt1 = jnp.zeros_like(m1)
    for c in range(C):
        l0 = cu(t0, c)
        l1 = cu(t1, c)
        se0 = se0 + jnp.exp(l0 - m0)
        se1 = se1 + jnp.exp(l1 - m1)
        isc = tgt == c
        gt0 = gt0 + jnp.where(isc, l0, 0.0)
        gt1 = gt1 + jnp.where(isc, l1, 0.0)

    lse0 = m0 + jnp.log(se0)
    lse1 = m1 + jnp.log(se1)

    valid = tgt != ignore_index
    validf = valid.astype(jnp.float32)

    ce0 = lse0 - gt0
    ce_ref[0] = ce0
    prob = jnp.where(valid, jnp.exp(-ce0), jnp.float32(_INVALID_PROB))
    prob_ref[0] = prob

    s2 = _sum2d((lse1 - gt1) * validf)
    c2 = _sum2d(validf)

    row = jax.lax.broadcasted_iota(jnp.int32, (8, 128), 0)
    col = jax.lax.broadcasted_iota(jnp.int32, (8, 128), 1)
    stats = jnp.zeros((8, 128), jnp.float32)
    stats = jnp.where((row == 0) & (col == 0), jnp.broadcast_to(s2, stats.shape), stats)
    stats = jnp.where((row == 0) & (col == 1), jnp.broadcast_to(c2, stats.shape), stats)
    for j, e in enumerate(edges):
        cnt = _sum2d(jnp.where(prob <= e, 1.0, 0.0))
        stats = jnp.where((row == 1) & (col == j), jnp.broadcast_to(cnt, stats.shape), stats)
    stats_ref[0, 0] = stats


def _select_kernel(thr_ref, ce_ref, prob_ref, out_ref):
    thr = thr_ref[0]
    kept = jnp.where(prob_ref[0] <= thr, 1.0, 0.0)
    s1 = _sum2d(ce_ref[0] * kept)
    c1 = _sum2d(kept)
    row = jax.lax.broadcasted_iota(jnp.int32, (8, 128), 0)
    col = jax.lax.broadcasted_iota(jnp.int32, (8, 128), 1)
    out = jnp.zeros((8, 128), jnp.float32)
    out = jnp.where((row == 0) & (col == 0), jnp.broadcast_to(s1, out.shape), out)
    out = jnp.where((row == 0) & (col == 1), jnp.broadcast_to(c1, out.shape), out)
    out_ref[0, 0] = out


def kernel(pred0, pred1, target):
    N, C, h, w = map(int, pred0.shape)
    H, W = int(target.shape[1]), int(target.shape[2])
    target = target.astype(jnp.int32)

    w_pad = _round_up(w, 128)
    W_pad = _round_up(W, 128)
    ntiles = _cdiv(H, _TILE_H)
    tH = _round_up(_cdiv(H, ntiles), 8)
    H_pad = tH * ntiles

    wh = jnp.zeros((H_pad, h), jnp.float32).at[:H].set(_interp_matrix(H, h))
    wwt = jnp.zeros((w_pad, W_pad), jnp.float32).at[:w, :W].set(_interp_matrix(W, w).T)
    wh = wh.astype(_MM_DTYPE)
    wwt = wwt.astype(_MM_DTYPE)

    def pack(p):
        p = jnp.transpose(p, (0, 2, 1, 3))
        p = jnp.pad(p, ((0, 0), (0, 0), (0, 0), (0, w_pad - w)))
        return p.reshape(N, h, C * w_pad).astype(_MM_DTYPE)

    p0r, p1r = pack(pred0), pack(pred1)
    tgt_p = jnp.pad(target, ((0, 0), (0, H_pad - H), (0, W_pad - W)),
                    constant_values=_IGNORE_INDEX)

    edges = _edges_tuple(_THRESH, _HIST_BINS)
    body = functools.partial(_main_kernel, num_classes=C, w_pad=w_pad,
                             ignore_index=_IGNORE_INDEX, edges=edges)
    ce, prob, stats = pl.pallas_call(
        body,
        out_shape=(jax.ShapeDtypeStruct((N, H_pad, W_pad), jnp.float32),
                   jax.ShapeDtypeStruct((N, H_pad, W_pad), jnp.float32),
                   jax.ShapeDtypeStruct((N, ntiles, 8, 128), jnp.float32)),
        grid_spec=pltpu.PrefetchScalarGridSpec(
            num_scalar_prefetch=0,
            grid=(N, ntiles),
            in_specs=[pl.BlockSpec((1, h, C * w_pad), lambda n, i: (n, 0, 0)),
                      pl.BlockSpec((1, h, C * w_pad), lambda n, i: (n, 0, 0)),
                      pl.BlockSpec((tH, h), lambda n, i: (i, 0)),
                      pl.BlockSpec((w_pad, W_pad), lambda n, i: (0, 0)),
                      pl.BlockSpec((1, tH, W_pad), lambda n, i: (n, i, 0))],
            out_specs=[pl.BlockSpec((1, tH, W_pad), lambda n, i: (n, i, 0)),
                       pl.BlockSpec((1, tH, W_pad), lambda n, i: (n, i, 0)),
                       pl.BlockSpec((1, 1, 8, 128), lambda n, i: (n, i, 0, 0))]),
        compiler_params=pltpu.CompilerParams(
            dimension_semantics=("parallel", "parallel"),
            vmem_limit_bytes=_VMEM_LIMIT),
    )(p0r, p1r, wh, wwt, tgt_p)

    s2 = jnp.sum(stats[:, :, 0, 0])
    c2 = jnp.sum(stats[:, :, 0, 1])
    cum = jnp.sum(stats[:, :, 1, :_HIST_BINS], axis=(0, 1))

    edges_arr = jnp.asarray(edges, jnp.float32)
    k = jnp.minimum(jnp.float32(_MIN_KEPT), c2)
    idx = jnp.argmax(cum >= k)
    thr = jnp.maximum(edges_arr[idx], jnp.float32(_THRESH))
    thr = jnp.where(jnp.float32(_MIN_KEPT) >= c2, jnp.float32(_KEEP_ALL_THR), thr)
    thr = thr.reshape(1).astype(jnp.float32)

    data_spec = lambda: pl.BlockSpec((1, tH, W_pad), lambda n, i: (n, i, 0))
    sel = pl.pallas_call(
        _select_kernel,
        out_shape=jax.ShapeDtypeStruct((N, ntiles, 8, 128), jnp.float32),
        grid_spec=pltpu.PrefetchScalarGridSpec(
            num_scalar_prefetch=0,
            grid=(N, ntiles),
            in_specs=[pl.BlockSpec(memory_space=pltpu.MemorySpace.SMEM),
                      data_spec(), data_spec()],
            out_specs=pl.BlockSpec((1, 1, 8, 128), lambda n, i: (n, i, 0, 0))),
        compiler_params=pltpu.CompilerParams(
            dimension_semantics=("parallel", "parallel"),
            vmem_limit_bytes=_VMEM_LIMIT),
    )(thr, ce, prob)

    s1 = jnp.sum(sel[:, :, 0, 0])
    c1 = jnp.sum(sel[:, :, 0, 1])
    loss1 = jnp.where(c1 > 0, s1 / jnp.maximum(c1, 1.0), 0.0)
    loss2 = jnp.where(c2 > 0, s2 / jnp.maximum(c2, 1.0), 0.0)
    return loss1 + 0.4 * loss2


# speedup vs baseline: 1.6616x; 1.3726x over previous
import functools

import jax
import jax.numpy as jnp
from jax.experimental import pallas as pl
from jax.experimental.pallas import tpu as pltpu

_IGNORE_INDEX = 255
_THRESH = 0.7
_MIN_KEPT = 100000
_HIST_BINS = 64
_GROUPS = 8
_KEEP_ALL_THR = 1.5
_INVALID_PROB = 2.0
_MM_DTYPE = jnp.bfloat16
_TILE_H = 256
_VMEM_LIMIT = (64 << 20) * 3 // 4


def _cdiv(a, b):
    return -(-a // b)


def _round_up(a, b):
    return _cdiv(a, b) * b


def _interp_matrix(out_size, in_size):
    if out_size == 1:
        src = jnp.zeros((1,), jnp.float32)
    else:
        src = jnp.arange(out_size, dtype=jnp.float32) * (in_size - 1) / (out_size - 1)
    i0 = jnp.clip(jnp.floor(src).astype(jnp.int32), 0, in_size - 1)
    i1 = jnp.clip(i0 + 1, 0, in_size - 1)
    w1 = src - i0.astype(jnp.float32)
    w0 = 1.0 - w1
    cols = jnp.arange(in_size, dtype=jnp.int32)[None, :]
    mat = (w0[:, None] * (cols == i0[:, None]).astype(jnp.float32)
           + w1[:, None] * (cols == i1[:, None]).astype(jnp.float32))
    return mat


def _edges_tuple(thresh, nbins):
    step = (1.0 - float(thresh)) / (nbins - 1)
    return tuple([float(thresh) + j * step for j in range(nbins - 1)] + [1.0 + 1e-3])


def _lanesum(x):
    R, W = x.shape
    y = x[:, :128]
    for g in range(1, W // 128):
        y = y + x[:, g * 128:(g + 1) * 128]
    return jnp.sum(y, axis=0, keepdims=True)


def _main_kernel(p0_ref, p1_ref, wh_ref, wwt_ref, tgt_ref,
                 ce_ref, prob_ref, stats_ref,
                 *, num_classes, w_pad, ignore_index, coarse_edges, stat_rows):
    C = num_classes
    wh = wh_ref[...]
    wwt = wwt_ref[...]
    tgt = tgt_ref[0]
    p0 = p0_ref[0]
    p1 = p1_ref[0]

    mm = lambda a, b: jnp.dot(a, b, preferred_element_type=jnp.float32)

    m0c = p0[:, :w_pad]
    m1c = p1[:, :w_pad]
    for c in range(1, C):
        m0c = jnp.maximum(m0c, p0[:, c * w_pad:(c + 1) * w_pad])
        m1c = jnp.maximum(m1c, p1[:, c * w_pad:(c + 1) * w_pad])

    t0 = mm(wh, jnp.concatenate([p0, m0c], axis=1)).astype(_MM_DTYPE)
    t1 = mm(wh, jnp.concatenate([p1, m1c], axis=1)).astype(_MM_DTYPE)
    tm0 = t0[:, C * w_pad:]
    tm1 = t1[:, C * w_pad:]

    cu = lambda t, tm, c: mm(t[:, c * w_pad:(c + 1) * w_pad] - tm, wwt)

    se0 = jnp.zeros((tgt.shape[0], tgt.shape[1]), jnp.float32)
    se1 = jnp.zeros_like(se0)
    gt0 = jnp.zeros_like(se0)
    gt1 = jnp.zeros_like(se0)
    for c in range(C):
        l0 = cu(t0, tm0, c)
        l1 = cu(t1, tm1, c)
        se0 = se0 + jnp.exp(l0)
        se1 = se1 + jnp.exp(l1)
        iscf = (tgt == c).astype(jnp.float32)
        gt0 = gt0 + l0 * iscf
        gt1 = gt1 + l1 * iscf

    ce0 = jnp.log(se0) - gt0
    ce1 = jnp.log(se1) - gt1

    valid = tgt != ignore_index
    validf = valid.astype(jnp.float32)

    ce_ref[0] = ce0
    prob = jnp.where(valid, jnp.exp(-ce0), jnp.float32(_INVALID_PROB))
    prob_ref[0] = prob

    parts = [_lanesum(jnp.where(prob <= e, 1.0, 0.0)) for e in coarse_edges]
    parts.append(_lanesum(ce1 * validf))
    parts.append(_lanesum(validf))
    while len(parts) < stat_rows:
        parts.append(jnp.zeros((1, 128), jnp.float32))
    stats_ref[0, 0] = jnp.concatenate(parts, axis=0)


def _select_kernel(cand_ref, ce_ref, prob_ref, out_ref, *, ncand, out_rows):
    ce = ce_ref[0]
    prob = prob_ref[0]
    parts = []
    for l in range(ncand):
        keptf = jnp.where(prob <= cand_ref[l], 1.0, 0.0)
        parts.append(_lanesum(keptf))
        parts.append(_lanesum(ce * keptf))
    while len(parts) < out_rows:
        parts.append(jnp.zeros((1, 128), jnp.float32))
    out_ref[0, 0] = jnp.concatenate(parts, axis=0)


def kernel(pred0, pred1, target):
    N, C, h, w = map(int, pred0.shape)
    H, W = int(target.shape[1]), int(target.shape[2])
    target = target.astype(jnp.int32)

    w_pad = _round_up(w, 128)
    W_pad = _round_up(W, 128)
    ntiles = _cdiv(H, _TILE_H)
    tH = _round_up(_cdiv(H, ntiles), 8)
    H_pad = tH * ntiles

    wh = jnp.zeros((H_pad, h), jnp.float32).at[:H].set(_interp_matrix(H, h))
    wwt = jnp.zeros((w_pad, W_pad), jnp.float32).at[:w, :W].set(_interp_matrix(W, w).T)
    wh = wh.astype(_MM_DTYPE)
    wwt = wwt.astype(_MM_DTYPE)

    def pack(p):
        p = jnp.transpose(p, (0, 2, 1, 3))
        p = jnp.pad(p, ((0, 0), (0, 0), (0, 0), (0, w_pad - w)))
        return p.reshape(N, h, C * w_pad).astype(_MM_DTYPE)

    p0r, p1r = pack(pred0), pack(pred1)
    tgt_p = jnp.pad(target, ((0, 0), (0, H_pad - H), (0, W_pad - W)),
                    constant_values=_IGNORE_INDEX)

    edges = _edges_tuple(_THRESH, _HIST_BINS)
    bins_per_group = _HIST_BINS // _GROUPS
    coarse_edges = tuple(edges[g * bins_per_group + bins_per_group - 1]
                         for g in range(_GROUPS))
    stat_rows = _round_up(_GROUPS + 2, 8)

    body = functools.partial(_main_kernel, num_classes=C, w_pad=w_pad,
                             ignore_index=_IGNORE_INDEX,
                             coarse_edges=coarse_edges, stat_rows=stat_rows)
    ce, prob, stats = pl.pallas_call(
        body,
        out_shape=(jax.ShapeDtypeStruct((N, H_pad, W_pad), jnp.float32),
                   jax.ShapeDtypeStruct((N, H_pad, W_pad), jnp.float32),
                   jax.ShapeDtypeStruct((N, ntiles, stat_rows, 128), jnp.float32)),
        grid_spec=pltpu.PrefetchScalarGridSpec(
            num_scalar_prefetch=0,
            grid=(N, ntiles),
            in_specs=[pl.BlockSpec((1, h, C * w_pad), lambda n, i: (n, 0, 0)),
                      pl.BlockSpec((1, h, C * w_pad), lambda n, i: (n, 0, 0)),
                      pl.BlockSpec((tH, h), lambda n, i: (i, 0)),
                      pl.BlockSpec((w_pad, W_pad), lambda n, i: (0, 0)),
                      pl.BlockSpec((1, tH, W_pad), lambda n, i: (n, i, 0))],
            out_specs=[pl.BlockSpec((1, tH, W_pad), lambda n, i: (n, i, 0)),
                       pl.BlockSpec((1, tH, W_pad), lambda n, i: (n, i, 0)),
                       pl.BlockSpec((1, 1, stat_rows, 128),
                                    lambda n, i: (n, i, 0, 0))]),
        compiler_params=pltpu.CompilerParams(
            dimension_semantics=("parallel", "parallel"),
            vmem_limit_bytes=_VMEM_LIMIT),
    )(p0r, p1r, wh, wwt, tgt_p)

    s2 = jnp.sum(stats[:, :, _GROUPS])
    c2 = jnp.sum(stats[:, :, _GROUPS + 1])
    cum_coarse = jnp.sum(stats[:, :, :_GROUPS], axis=(0, 1, 3))

    edges_arr = jnp.asarray(edges, jnp.float32)
    k = jnp.minimum(jnp.float32(_MIN_KEPT), c2)
    grp = jnp.argmax(cum_coarse >= k)
    cand = jax.lax.dynamic_slice(edges_arr, (grp * bins_per_group,),
                                 (bins_per_group,))
    keep_all = jnp.float32(_MIN_KEPT) >= c2
    cand = jnp.where(keep_all, jnp.full_like(cand, _KEEP_ALL_THR), cand)
    cand = cand.astype(jnp.float32)

    out_rows = _round_up(2 * bins_per_group, 8)
    sel_body = functools.partial(_select_kernel, ncand=bins_per_group,
                                 out_rows=out_rows)
    data_spec = lambda: pl.BlockSpec((1, tH, W_pad), lambda n, i: (n, i, 0))
    sel = pl.pallas_call(
        sel_body,
        out_shape=jax.ShapeDtypeStruct((N, ntiles, out_rows, 128), jnp.float32),
        grid_spec=pltpu.PrefetchScalarGridSpec(
            num_scalar_prefetch=0,
            grid=(N, ntiles),
            in_specs=[pl.BlockSpec(memory_space=pltpu.MemorySpace.SMEM),
                      data_spec(), data_spec()],
            out_specs=pl.BlockSpec((1, 1, out_rows, 128),
                                   lambda n, i: (n, i, 0, 0))),
        compiler_params=pltpu.CompilerParams(
            dimension_semantics=("parallel", "parallel"),
            vmem_limit_bytes=_VMEM_LIMIT),
    )(cand, ce, prob)

    cnt_fine = jnp.sum(sel[:, :, 0:2 * bins_per_group:2], axis=(0, 1, 3))
    ces_fine = jnp.sum(sel[:, :, 1:2 * bins_per_group:2], axis=(0, 1, 3))

    l_idx = jnp.argmax(cnt_fine >= k)
    s1 = ces_fine[l_idx]
    c1 = cnt_fine[l_idx]

    loss1 = jnp.where(c1 > 0, s1 / jnp.maximum(c1, 1.0), 0.0)
    loss2 = jnp.where(c2 > 0, s2 / jnp.maximum(c2, 1.0), 0.0)
    return loss1 + 0.4 * loss2
